# Initial kernel scaffold; baseline (speedup 1.0000x reference)
import jax
import jax.numpy as jnp
from jax.experimental import pallas as pl


def kernel(x, ei, batch, W1, b1, g1, be1, W2, b2, g2, be2, W3, b3, g3, be3):
    raise NotImplementedError("write your pallas kernel here")


# speedup vs baseline: 1.0006x; 1.0006x over previous
import functools
import jax
import jax.numpy as jnp
from jax.experimental import pallas as pl

_NB = 2000


def _mm_body(x_ref, w_ref, o_ref):
    o_ref[...] = jnp.dot(x_ref[...], w_ref[...],
                         preferred_element_type=jnp.float32)


def _mm(x, w):
    n, k = x.shape
    f = w.shape[1]
    return pl.pallas_call(
        _mm_body,
        grid=(n // _NB,),
        in_specs=[pl.BlockSpec((_NB, k), lambda i: (i, 0)),
                  pl.BlockSpec((k, f), lambda i: (0, 0))],
        out_specs=pl.BlockSpec((_NB, f), lambda i: (i, 0)),
        out_shape=jax.ShapeDtypeStruct((n, f), jnp.float32),
    )(x, w)


def _stats_body(c_ref, o_ref):
    @pl.when(pl.program_id(0) == 0)
    def _():
        o_ref[...] = jnp.zeros_like(o_ref)
    c = c_ref[...]
    o_ref[0, :] += jnp.sum(c, axis=0)
    o_ref[1, :] += jnp.sum(c * c, axis=0)


def _stats(c):
    n, f = c.shape
    return pl.pallas_call(
        _stats_body,
        grid=(n // _NB,),
        in_specs=[pl.BlockSpec((_NB, f), lambda i: (i, 0))],
        out_specs=pl.BlockSpec((2, f), lambda i: (0, 0)),
        out_shape=jax.ShapeDtypeStruct((2, f), jnp.float32),
    )(c)


def _bnmm_body(n, c_ref, s_ref, g_ref, be_ref, w_ref, o_ref):
    mu = s_ref[0, :] / n
    var = s_ref[1, :] / n - mu * mu
    rstd = jax.lax.rsqrt(var + 1e-5)
    h = (c_ref[...] - mu[None, :]) * (rstd * g_ref[0, :])[None, :] \
        + be_ref[0, :][None, :]
    h = jnp.maximum(h, 0.0)
    o_ref[...] = jnp.dot(h, w_ref[...], preferred_element_type=jnp.float32)


def _bnmm(c, s, g, be, w):
    n, f = c.shape
    fo = w.shape[1]
    return pl.pallas_call(
        functools.partial(_bnmm_body, float(n)),
        grid=(n // _NB,),
        in_specs=[pl.BlockSpec((_NB, f), lambda i: (i, 0)),
                  pl.BlockSpec((2, f), lambda i: (0, 0)),
                  pl.BlockSpec((1, f), lambda i: (0, 0)),
                  pl.BlockSpec((1, f), lambda i: (0, 0)),
                  pl.BlockSpec((f, fo), lambda i: (0, 0))],
        out_specs=pl.BlockSpec((_NB, fo), lambda i: (i, 0)),
        out_shape=jax.ShapeDtypeStruct((n, fo), jnp.float32),
    )(c, s, g.reshape(1, f), be.reshape(1, f), w)


def _pool_body(g_count, h_ref, b_ref, o_ref):
    @pl.when(pl.program_id(0) == 0)
    def _():
        o_ref[...] = jnp.zeros_like(o_ref)
    h = h_ref[...]
    ones = jnp.ones((h.shape[0], 1), jnp.float32)
    ha = jnp.concatenate([h, ones], axis=1)
    gid = jax.lax.broadcasted_iota(jnp.int32, (h.shape[0], g_count), 1)
    onehot = (b_ref[...] == gid).astype(jnp.float32)
    o_ref[...] += jax.lax.dot_general(
        onehot, ha, (((0,), (0,)), ((), ())),
        preferred_element_type=jnp.float32)


def _pool(h, batch, g_count):
    n, f = h.shape
    return pl.pallas_call(
        functools.partial(_pool_body, g_count),
        grid=(n // _NB,),
        in_specs=[pl.BlockSpec((_NB, f), lambda i: (i, 0)),
                  pl.BlockSpec((_NB, 1), lambda i: (i, 0))],
        out_specs=pl.BlockSpec((g_count, f + 1), lambda i: (0, 0)),
        out_shape=jax.ShapeDtypeStruct((g_count, f + 1), jnp.float32),
    )(h, batch.astype(jnp.int32).reshape(n, 1))


def kernel(x, ei, batch, W1, b1, g1, be1, W2, b2, g2, be2, W3, b3, g3, be3):
    n = x.shape[0]
    g_count = 1024
    loop = jnp.arange(n, dtype=ei.dtype)
    src = jnp.concatenate([ei[0], loop])
    dst = jnp.concatenate([ei[1], loop])
    deg = jnp.zeros((n,), jnp.float32).at[dst].add(1.0)
    dinv = jnp.where(deg > 0, jax.lax.rsqrt(deg), 0.0)
    norm = dinv[src] * dinv[dst]

    def mp(m):
        msg = m[src] * norm[:, None]
        return jnp.zeros((n, m.shape[1]), m.dtype).at[dst].add(msg)

    m1 = _mm(x, W1)
    c1 = mp(m1)
    s1 = _stats(c1)
    m2 = _bnmm(c1, s1, g1, be1, W2)
    c2 = mp(m2)
    s2 = _stats(c2)
    m3 = _bnmm(c2, s2, g2, be2, W3)
    c3 = mp(m3)
    s3 = _stats(c3)
    h3 = _bnmm(c3, s3, g3, be3, jnp.eye(128, dtype=jnp.float32))
    pooled = _pool(h3, batch, g_count)
    return pooled[:, :128] / jnp.maximum(pooled[:, 128:], 1.0)
